# baseline (speedup 1.0000x reference)
import functools

import jax
import jax.numpy as jnp
from jax import lax
from jax.experimental import pallas as pl
from jax.experimental.pallas import tpu as pltpu
from jax.experimental.pallas import tpu_sc as plsc

R, C = 16384, 200
V, D = 2000, 9
N = R * C
L = 16
NC, NS = 2, 16
NW = NC * NS
COLS = R // NW
NCHUNK = 4
CH = COLS // NCHUNK
KS = CH // L
V_PAD = 2048
RS_ITERS = V // L


def _sc_partial_sums(movies_t, table_t):
    mesh = plsc.VectorSubcoreMesh(core_axis_name="c", subcore_axis_name="s")

    @functools.partial(
        pl.kernel, mesh=mesh,
        out_type=jax.ShapeDtypeStruct((NW, L), jnp.float32),
        compiler_params=pltpu.CompilerParams(needs_layout_passes=False),
        scratch_types=[
            pltpu.VMEM((C, CH), jnp.int32),
            pltpu.VMEM((C, CH), jnp.int32),
            pltpu.VMEM((D, V), jnp.float32),
            pltpu.VMEM((V_PAD,), jnp.float32),
            pltpu.VMEM((L,), jnp.float32),
            pltpu.SemaphoreType.DMA,
            pltpu.SemaphoreType.DMA,
        ],
    )
    def k(mov_hbm, tbl_hbm, out_hbm, mov_a, mov_b, tbl_v, rowsum_v, acc_v,
          sem_a, sem_b):
        wid = lax.axis_index("s") * NC + lax.axis_index("c")
        base = wid * COLS
        bufs = (mov_a, mov_b)
        sems = (sem_a, sem_b)

        def fetch(c):
            return pltpu.async_copy(
                mov_hbm.at[:, pl.ds(base + c * CH, CH)], bufs[c % 2],
                sems[c % 2])

        h = fetch(0)
        pltpu.sync_copy(tbl_hbm, tbl_v)

        def rs_body(b, _):
            acc = tbl_v[0, pl.ds(b * L, L)]
            for kk in range(1, D):
                acc = acc + tbl_v[kk, pl.ds(b * L, L)]
            rowsum_v[pl.ds(b * L, L)] = acc
            return 0

        lax.fori_loop(0, RS_ITERS, rs_body, 0)

        def gather_chunk(mov_v, accs):
            @plsc.parallel_loop(0, C, carry=accs, unroll=2)
            def accs_out(r, accs):
                accs = list(accs)
                for kk in range(KS):
                    idx = mov_v[r, pl.ds(kk * L, L)]
                    g = plsc.load_gather(rowsum_v, [idx])
                    accs[kk % 4] = accs[kk % 4] + g
                return tuple(accs)
            return accs_out

        zero = jnp.zeros((L,), jnp.float32)
        accs = (zero,) * 4
        for c in range(NCHUNK):
            h.wait()
            if c + 1 < NCHUNK:
                h = fetch(c + 1)
            accs = gather_chunk(bufs[c % 2], accs)

        a0, a1, a2, a3 = accs
        acc_v[...] = (a0 + a1) + (a2 + a3)
        pltpu.sync_copy(acc_v, out_hbm.at[wid])

    return k(movies_t, table_t)


def _tc_finish(partials):
    def body(p_ref, o_ref):
        o_ref[0, 0] = jax.nn.sigmoid(jnp.sum(p_ref[...]) * (1.0 / (N * D)))

    return pl.pallas_call(
        body,
        out_shape=jax.ShapeDtypeStruct((1, 1), jnp.float32),
        out_specs=pl.BlockSpec(memory_space=pltpu.SMEM),
    )(partials)


def kernel(movies, ratings, table):
    del ratings
    partials = _sc_partial_sums(movies.T, table.T)
    return _tc_finish(partials)[0, 0]
